# Initial kernel scaffold; baseline (speedup 1.0000x reference)
import jax
import jax.numpy as jnp
from jax.experimental import pallas as pl


def kernel(adj_matrix, dep_rel_matrix, dep_table):
    raise NotImplementedError("write your pallas kernel here")


# speedup vs baseline: 2.3478x; 2.3478x over previous
import jax
import jax.numpy as jnp
from jax import lax
from jax.experimental import pallas as pl
from jax.experimental.pallas import tpu as pltpu
from jax.experimental.pallas import tpu_sc as plsc

DEP_VOCAB = 50
EMBED_DIM = 64
B, S = 8, 256
N = B * S * S
NC, NS = 2, 16
NW = NC * NS
PER_W = N // NW
CHUNK = 512
NCHUNK = PER_W // CHUNK
GROUPS = CHUNK // 16
LANES = 16


def _sc_body(idx_hbm, adj_hbm, tab_hbm, out_hbm, tab_v, idx_v, adj_v, out_v):
    wid = lax.axis_index("s") * NC + lax.axis_index("c")
    base = wid * PER_W
    pltpu.sync_copy(tab_hbm, tab_v)

    iota = lax.iota(jnp.int32, LANES)
    iota64 = iota * EMBED_DIM
    perms = [(iota + s) & (LANES - 1) for s in range(LANES)]

    def chunk_body(ci, carry):
        cbase = base + ci * CHUNK
        pltpu.sync_copy(idx_hbm.at[pl.ds(cbase, CHUNK)], idx_v)
        pltpu.sync_copy(adj_hbm.at[pl.ds(cbase, CHUNK)], adj_v)

        def group_body(g, c2):
            off = g * LANES
            idx16 = idx_v[pl.ds(off, LANES)]
            adj16 = adj_v[pl.ds(off, LANES)]
            idx64 = idx16 * EMBED_DIM
            rowb = iota64 + off * EMBED_DIM
            for s in range(LANES):
                idxp = idx64 + perms[s]
                rowp = rowb + perms[s]
                for c in range(4):
                    r = plsc.load_gather(tab_v, [idxp + c * LANES])
                    plsc.store_scatter(out_v, [rowp + c * LANES], r * adj16)
            return c2

        lax.fori_loop(0, GROUPS, group_body, 0)
        pltpu.sync_copy(out_v, out_hbm.at[pl.ds(cbase * EMBED_DIM,
                                                CHUNK * EMBED_DIM)])
        return carry

    lax.fori_loop(0, NCHUNK, chunk_body, 0)


@jax.jit
def _sc_call(idx, adjf, tab):
    mesh = plsc.VectorSubcoreMesh(core_axis_name="c", subcore_axis_name="s",
                                  num_cores=NC, num_subcores=NS)
    fn = pl.kernel(
        _sc_body,
        out_type=jax.ShapeDtypeStruct((N * EMBED_DIM,), jnp.float32),
        mesh=mesh,
        compiler_params=pltpu.CompilerParams(needs_layout_passes=False),
        scratch_types=[
            pltpu.VMEM((DEP_VOCAB * EMBED_DIM,), jnp.float32),
            pltpu.VMEM((CHUNK,), jnp.int32),
            pltpu.VMEM((CHUNK,), jnp.float32),
            pltpu.VMEM((CHUNK * EMBED_DIM,), jnp.float32),
        ],
    )
    return fn(idx, adjf, tab)


def kernel(adj_matrix, dep_rel_matrix, dep_table):
    idx = dep_rel_matrix.reshape(-1).astype(jnp.int32)
    adjf = adj_matrix.reshape(-1).astype(jnp.float32)
    tab = dep_table.reshape(-1).astype(jnp.float32)
    out = _sc_call(idx, adjf, tab)
    return out.reshape(B, S, S, EMBED_DIM)


# speedup vs baseline: 3.2395x; 1.3798x over previous
import jax
import jax.numpy as jnp
from jax import lax
from jax.experimental import pallas as pl
from jax.experimental.pallas import tpu as pltpu
from jax.experimental.pallas import tpu_sc as plsc

DEP_VOCAB = 50
EMBED_DIM = 64
B, S = 8, 256
N = B * S * S
NC, NS = 2, 16
NW = NC * NS
NROWS = B * S
ROWS_PER_W = NROWS // NW
ROW_WORDS = EMBED_DIM * S
CHUNK_ROWS = 2
CHUNK = CHUNK_ROWS * S
NCHUNK = ROWS_PER_W // CHUNK_ROWS
LANES = 16


def _sc_body(idx_hbm, adj_hbm, tab_hbm, out_hbm, tab_v, idx_v, adj_v, out_v):
    wid = lax.axis_index("s") * NC + lax.axis_index("c")
    row0 = wid * ROWS_PER_W
    pltpu.sync_copy(tab_hbm, tab_v)

    iota = lax.iota(jnp.int32, LANES)
    perms = [(iota + s) & (LANES - 1) for s in range(LANES)]
    perms256 = [p * S for p in perms]

    def chunk_body(ci, carry):
        rbase = row0 + ci * CHUNK_ROWS
        pltpu.sync_copy(idx_hbm.at[pl.ds(rbase * S, CHUNK)], idx_v)
        pltpu.sync_copy(adj_hbm.at[pl.ds(rbase * S, CHUNK)], adj_v)

        def group_body(g, c2):
            off = g * LANES
            idx16 = idx_v[pl.ds(off, LANES)]
            adj16 = adj_v[pl.ds(off, LANES)]
            idx64 = idx16 * EMBED_DIM
            jb = iota + ((g & ~15) << 10 | (g & 15) << 4)
            for s in range(LANES):
                idxp = idx64 + perms[s]
                sa0 = jb + perms256[s]
                for c in range(4):
                    r = plsc.load_gather(tab_v, [idxp + c * LANES])
                    plsc.store_scatter(out_v, [sa0 + c * LANES * S],
                                       r * adj16)
            return c2

        lax.fori_loop(0, CHUNK * 4 // EMBED_DIM, group_body, 0)
        pltpu.sync_copy(out_v, out_hbm.at[pl.ds(rbase * ROW_WORDS,
                                                CHUNK_ROWS * ROW_WORDS)])
        return carry

    lax.fori_loop(0, NCHUNK, chunk_body, 0)


@jax.jit
def _sc_call(idx, adjf, tab):
    mesh = plsc.VectorSubcoreMesh(core_axis_name="c", subcore_axis_name="s",
                                  num_cores=NC, num_subcores=NS)
    fn = pl.kernel(
        _sc_body,
        out_type=jax.ShapeDtypeStruct((N * EMBED_DIM,), jnp.float32),
        mesh=mesh,
        compiler_params=pltpu.CompilerParams(needs_layout_passes=False),
        scratch_types=[
            pltpu.VMEM((DEP_VOCAB * EMBED_DIM,), jnp.float32),
            pltpu.VMEM((CHUNK,), jnp.int32),
            pltpu.VMEM((CHUNK,), jnp.float32),
            pltpu.VMEM((CHUNK_ROWS * ROW_WORDS,), jnp.float32),
        ],
    )
    return fn(idx, adjf, tab)


def kernel(adj_matrix, dep_rel_matrix, dep_table):
    idx = dep_rel_matrix.reshape(-1).astype(jnp.int32)
    adjf = adj_matrix.reshape(-1).astype(jnp.float32)
    tab = dep_table.reshape(-1).astype(jnp.float32)
    out = _sc_call(idx, adjf, tab)
    return out.reshape(B, S, EMBED_DIM, S).transpose(0, 1, 3, 2)
